# baseline (speedup 1.0000x reference)
import jax
import jax.numpy as jnp
from jax.experimental import pallas as pl
from jax.experimental.pallas import tpu as pltpu

N = 4096
IN_F = 128
OUT_F = 128
ALPHA = 0.2
BM = 512
NB = N // BM
DEPTH = 3
HA = 256
LOG2E = 1.4426950408889634


def _gat_kernel(x_ref, w_ref, a1_ref, a2r_ref, adj_hbm, out_ref,
                buf, sems, hb1_s, srcs_s, dstt_s, dmax_s, meanh_s):
    def start(k, slot):
        pltpu.make_async_copy(
            adj_hbm.at[pl.ds(k * BM, BM), :], buf.at[slot], sems.at[slot]
        ).start()

    for k in range(DEPTH):
        start(k, k)

    h = jnp.dot(x_ref[...], w_ref[...], preferred_element_type=jnp.float32)
    hb1_s[:, :OUT_F] = h.astype(jnp.bfloat16)
    hb1_s[:, OUT_F:OUT_F + 1] = jnp.ones((N, 1), jnp.bfloat16)
    hb1_s[:, OUT_F + 1:] = jnp.zeros((N, HA - OUT_F - 1), jnp.bfloat16)
    meanh_s[...] = jnp.mean(h, axis=0, keepdims=True)
    srcs_s[...] = jnp.dot(h, a1_ref[...],
                          preferred_element_type=jnp.float32) * LOG2E
    dstt = jax.lax.dot_general(
        a2r_ref[...], h, (((1,), (1,)), ((), ())),
        preferred_element_type=jnp.float32) * LOG2E
    dstt_s[...] = dstt
    dmax_s[...] = jnp.max(dstt).reshape(1, 1)

    def body(k, carry):
        slot = jax.lax.rem(k, DEPTH)
        pltpu.make_async_copy(
            adj_hbm.at[pl.ds(k * BM, BM), :], buf.at[slot], sems.at[slot]
        ).wait()

        srcs = srcs_s[pl.ds(k * BM, BM), :]
        t = srcs + dmax_s[0, 0]
        m = jnp.maximum(t, ALPHA * t)
        sa = srcs - m
        sb = ALPHA * srcs - m
        dstts = dstt_s[...]
        dstts2 = ALPHA * dstts
        e = jnp.exp2(jnp.maximum(sa + dstts, sb + dstts2)) * buf[slot]
        s = jnp.sum(e, axis=1, keepdims=True)
        hp = jnp.dot(e.astype(jnp.bfloat16), hb1_s[:, :OUT_F],
                     preferred_element_type=jnp.float32)
        s_safe = jnp.where(s > 0, s, 1.0)
        hp = jnp.where(s > 0, hp / s_safe, meanh_s[...])
        out_ref[pl.ds(k * BM, BM), :] = jnp.where(
            hp > 0, hp, jnp.exp(jnp.minimum(hp, 0.0)) - 1.0)

        @pl.when(k + DEPTH < NB)
        def _():
            start(k + DEPTH, slot)

        return carry

    jax.lax.fori_loop(0, NB, body, 0)


@jax.jit
def kernel(input, adj, W, a):
    a1 = a[:OUT_F].reshape(IN_F, 1)
    a2r = a[OUT_F:].reshape(1, IN_F)
    out = pl.pallas_call(
        _gat_kernel,
        in_specs=[
            pl.BlockSpec(memory_space=pltpu.MemorySpace.VMEM),
            pl.BlockSpec(memory_space=pltpu.MemorySpace.VMEM),
            pl.BlockSpec(memory_space=pltpu.MemorySpace.VMEM),
            pl.BlockSpec(memory_space=pltpu.MemorySpace.VMEM),
            pl.BlockSpec(memory_space=pl.ANY),
        ],
        out_specs=pl.BlockSpec(memory_space=pltpu.MemorySpace.VMEM),
        out_shape=jax.ShapeDtypeStruct((N, OUT_F), jnp.float32),
        scratch_shapes=[
            pltpu.VMEM((DEPTH, BM, N), jnp.float32),
            pltpu.SemaphoreType.DMA((DEPTH,)),
            pltpu.VMEM((N, HA), jnp.bfloat16),
            pltpu.VMEM((N, 1), jnp.float32),
            pltpu.VMEM((1, N), jnp.float32),
            pltpu.VMEM((1, 1), jnp.float32),
            pltpu.VMEM((1, OUT_F), jnp.float32),
        ],
    )(input, W, a1, a2r, adj)
    return out


# speedup vs baseline: 1.1017x; 1.1017x over previous
import jax
import jax.numpy as jnp
from jax.experimental import pallas as pl
from jax.experimental.pallas import tpu as pltpu

N = 4096
IN_F = 128
OUT_F = 128
ALPHA = 0.2
BM = 512
NB = N // BM
DEPTH = 4
HA = 256
LOG2E = 1.4426950408889634


def _gat_kernel(x_ref, w_ref, a1_ref, a2r_ref, adj_hbm, out_ref,
                buf, sems, hb1_s, srcs_s, dstt_s, dmax_s, meanh_s):
    def start(k, slot):
        pltpu.make_async_copy(
            adj_hbm.at[pl.ds(k * BM, BM), :], buf.at[slot], sems.at[slot]
        ).start()

    for k in range(DEPTH):
        start(k, k)

    h = jnp.dot(x_ref[...], w_ref[...], preferred_element_type=jnp.float32)
    hb1_s[:, :OUT_F] = h.astype(jnp.bfloat16)
    hb1_s[:, OUT_F:OUT_F + 1] = jnp.ones((N, 1), jnp.bfloat16)
    hb1_s[:, OUT_F + 1:] = jnp.zeros((N, HA - OUT_F - 1), jnp.bfloat16)
    meanh_s[...] = jnp.mean(h, axis=0, keepdims=True)
    srcs_s[...] = jnp.dot(h, a1_ref[...],
                          preferred_element_type=jnp.float32) * LOG2E
    dstt = jax.lax.dot_general(
        a2r_ref[...], h, (((1,), (1,)), ((), ())),
        preferred_element_type=jnp.float32) * LOG2E
    dstt_s[...] = dstt
    dmax_s[...] = jnp.max(dstt).reshape(1, 1)

    def body(k, carry):
        slot = jax.lax.rem(k, DEPTH)
        pltpu.make_async_copy(
            adj_hbm.at[pl.ds(k * BM, BM), :], buf.at[slot], sems.at[slot]
        ).wait()

        srcs = srcs_s[pl.ds(k * BM, BM), :]
        t = srcs + dmax_s[0, 0]
        m = jnp.maximum(t, ALPHA * t)
        sa = srcs - m
        sb = ALPHA * srcs - m
        dstts = dstt_s[...]
        dstts2 = ALPHA * dstts
        e = jnp.exp2(jnp.maximum(sa + dstts, sb + dstts2)) * buf[slot]
        hp1 = jnp.dot(e.astype(jnp.bfloat16), hb1_s[...],
                      preferred_element_type=jnp.float32)
        s = hp1[:, OUT_F:OUT_F + 1]
        hp = hp1[:, :OUT_F]
        s_safe = jnp.where(s > 0, s, 1.0)
        hp = jnp.where(s > 0, hp / s_safe, meanh_s[...])
        out_ref[pl.ds(k * BM, BM), :] = jnp.where(
            hp > 0, hp, jnp.exp(jnp.minimum(hp, 0.0)) - 1.0)

        @pl.when(k + DEPTH < NB)
        def _():
            start(k + DEPTH, slot)

        return carry

    jax.lax.fori_loop(0, NB, body, 0)


@jax.jit
def kernel(input, adj, W, a):
    a1 = a[:OUT_F].reshape(IN_F, 1)
    a2r = a[OUT_F:].reshape(1, IN_F)
    out = pl.pallas_call(
        _gat_kernel,
        in_specs=[
            pl.BlockSpec(memory_space=pltpu.MemorySpace.VMEM),
            pl.BlockSpec(memory_space=pltpu.MemorySpace.VMEM),
            pl.BlockSpec(memory_space=pltpu.MemorySpace.VMEM),
            pl.BlockSpec(memory_space=pltpu.MemorySpace.VMEM),
            pl.BlockSpec(memory_space=pl.ANY),
        ],
        out_specs=pl.BlockSpec(memory_space=pltpu.MemorySpace.VMEM),
        out_shape=jax.ShapeDtypeStruct((N, OUT_F), jnp.float32),
        scratch_shapes=[
            pltpu.VMEM((DEPTH, BM, N), jnp.float32),
            pltpu.SemaphoreType.DMA((DEPTH,)),
            pltpu.VMEM((N, HA), jnp.bfloat16),
            pltpu.VMEM((N, 1), jnp.float32),
            pltpu.VMEM((1, N), jnp.float32),
            pltpu.VMEM((1, 1), jnp.float32),
            pltpu.VMEM((1, OUT_F), jnp.float32),
        ],
    )(input, W, a1, a2r, adj)
    return out


# speedup vs baseline: 1.1251x; 1.0212x over previous
import jax
import jax.numpy as jnp
from jax.experimental import pallas as pl
from jax.experimental.pallas import tpu as pltpu

N = 4096
IN_F = 128
OUT_F = 128
ALPHA = 0.2
BM = 256
NB = N // BM
DEPTH = 6
HA = 256
LOG2E = 1.4426950408889634


def _gat_kernel(x_ref, w_ref, a1_ref, a2r_ref, adj_hbm, out_ref,
                buf, sems, hb1_s, srcs_s, dstt_s, dmax_s, meanh_s):
    def start(k, slot):
        pltpu.make_async_copy(
            adj_hbm.at[pl.ds(k * BM, BM), :], buf.at[slot], sems.at[slot]
        ).start()

    for k in range(DEPTH):
        start(k, k)

    h = jnp.dot(x_ref[...], w_ref[...], preferred_element_type=jnp.float32)
    hb1_s[:, :OUT_F] = h.astype(jnp.bfloat16)
    hb1_s[:, OUT_F:OUT_F + 1] = jnp.ones((N, 1), jnp.bfloat16)
    hb1_s[:, OUT_F + 1:] = jnp.zeros((N, HA - OUT_F - 1), jnp.bfloat16)
    meanh_s[...] = jnp.mean(h, axis=0, keepdims=True)
    srcs_s[...] = jnp.dot(h, a1_ref[...],
                          preferred_element_type=jnp.float32) * LOG2E
    dstt = jax.lax.dot_general(
        a2r_ref[...], h, (((1,), (1,)), ((), ())),
        preferred_element_type=jnp.float32) * LOG2E
    dstt_s[...] = dstt
    dmax_s[...] = jnp.max(dstt).reshape(1, 1)

    def body(k, carry):
        slot = jax.lax.rem(k, DEPTH)
        pltpu.make_async_copy(
            adj_hbm.at[pl.ds(k * BM, BM), :], buf.at[slot], sems.at[slot]
        ).wait()

        srcs = srcs_s[pl.ds(k * BM, BM), :]
        t = srcs + dmax_s[0, 0]
        m = jnp.maximum(t, ALPHA * t)
        sa = srcs - m
        sb = ALPHA * srcs - m
        dstts = dstt_s[...]
        dstts2 = ALPHA * dstts
        e = jnp.exp2(jnp.maximum(sa + dstts, sb + dstts2)) * buf[slot]
        hp1 = jnp.dot(e.astype(jnp.bfloat16), hb1_s[...],
                      preferred_element_type=jnp.float32)
        s = hp1[:, OUT_F:OUT_F + 1]
        hp = hp1[:, :OUT_F]
        s_safe = jnp.where(s > 0, s, 1.0)
        hp = jnp.where(s > 0, hp / s_safe, meanh_s[...])
        out_ref[pl.ds(k * BM, BM), :] = jnp.where(
            hp > 0, hp, jnp.exp(jnp.minimum(hp, 0.0)) - 1.0)

        @pl.when(k + DEPTH < NB)
        def _():
            start(k + DEPTH, slot)

        return carry

    jax.lax.fori_loop(0, NB, body, 0)


@jax.jit
def kernel(input, adj, W, a):
    a1 = a[:OUT_F].reshape(IN_F, 1)
    a2r = a[OUT_F:].reshape(1, IN_F)
    out = pl.pallas_call(
        _gat_kernel,
        in_specs=[
            pl.BlockSpec(memory_space=pltpu.MemorySpace.VMEM),
            pl.BlockSpec(memory_space=pltpu.MemorySpace.VMEM),
            pl.BlockSpec(memory_space=pltpu.MemorySpace.VMEM),
            pl.BlockSpec(memory_space=pltpu.MemorySpace.VMEM),
            pl.BlockSpec(memory_space=pl.ANY),
        ],
        out_specs=pl.BlockSpec(memory_space=pltpu.MemorySpace.VMEM),
        out_shape=jax.ShapeDtypeStruct((N, OUT_F), jnp.float32),
        scratch_shapes=[
            pltpu.VMEM((DEPTH, BM, N), jnp.float32),
            pltpu.SemaphoreType.DMA((DEPTH,)),
            pltpu.VMEM((N, HA), jnp.bfloat16),
            pltpu.VMEM((N, 1), jnp.float32),
            pltpu.VMEM((1, N), jnp.float32),
            pltpu.VMEM((1, 1), jnp.float32),
            pltpu.VMEM((1, OUT_F), jnp.float32),
        ],
    )(input, W, a1, a2r, adj)
    return out
